# baseline (speedup 1.0000x reference)
import functools

import jax
import jax.numpy as jnp
from jax import lax
from jax.experimental import pallas as pl
from jax.experimental.pallas import tpu as pltpu
from jax.experimental.pallas import tpu_sc as plsc

H = 128
NNODES = {"cpg": 100000, "gene": 20000, "mirna": 2000}
RELSPEC = [("maps_to", "cpg", "gene", False),
           ("targets", "mirna", "gene", False),
           ("ppi", "gene", "gene", True),
           ("rev_maps_to", "gene", "cpg", False),
           ("rev_targets", "gene", "mirna", False)]

_EPS_SM = 1e-16
_EPS_LN = 1e-5


def _rup(x, m):
    return (x + m - 1) // m * m


_BLKP = 512


def _proj_in(x, W, b):
    n = x.shape[0]

    def body(x_ref, w_ref, b_ref, y_ref):
        y_ref[...] = jnp.dot(x_ref[...], w_ref[...],
                             preferred_element_type=jnp.float32) + b_ref[...]

    return pl.pallas_call(
        body,
        grid=(pl.cdiv(n, _BLKP),),
        in_specs=[pl.BlockSpec((_BLKP, H), lambda i: (i, 0)),
                  pl.BlockSpec((H, H), lambda i: (0, 0)),
                  pl.BlockSpec((1, H), lambda i: (0, 0))],
        out_specs=pl.BlockSpec((_BLKP, H), lambda i: (i, 0)),
        out_shape=jax.ShapeDtypeStruct((n, H), jnp.float32),
    )(x, W, b.reshape(1, H))


def _proj_full(x, W, a_s, a_d):
    n = x.shape[0]

    def body(x_ref, w_ref, as_ref, ad_ref, y_ref, als_ref, ald_ref):
        y = jnp.dot(x_ref[...], w_ref[...], preferred_element_type=jnp.float32)
        y_ref[...] = y
        als_ref[...] = jnp.sum(y * as_ref[...], axis=-1)
        ald_ref[...] = jnp.sum(y * ad_ref[...], axis=-1)

    return pl.pallas_call(
        body,
        grid=(pl.cdiv(n, _BLKP),),
        in_specs=[pl.BlockSpec((_BLKP, H), lambda i: (i, 0)),
                  pl.BlockSpec((H, H), lambda i: (0, 0)),
                  pl.BlockSpec((1, H), lambda i: (0, 0)),
                  pl.BlockSpec((1, H), lambda i: (0, 0))],
        out_specs=[pl.BlockSpec((_BLKP, H), lambda i: (i, 0)),
                   pl.BlockSpec((_BLKP,), lambda i: (i,)),
                   pl.BlockSpec((_BLKP,), lambda i: (i,))],
        out_shape=[jax.ShapeDtypeStruct((n, H), jnp.float32),
                   jax.ShapeDtypeStruct((n,), jnp.float32),
                   jax.ShapeDtypeStruct((n,), jnp.float32)],
    )(x, W, a_s.reshape(1, H), a_d.reshape(1, H))


def _proj_al(x, W, a):
    n = x.shape[0]

    def body(x_ref, w_ref, a_ref, al_ref):
        y = jnp.dot(x_ref[...], w_ref[...], preferred_element_type=jnp.float32)
        al_ref[...] = jnp.sum(y * a_ref[...], axis=-1)

    return pl.pallas_call(
        body,
        grid=(pl.cdiv(n, _BLKP),),
        in_specs=[pl.BlockSpec((_BLKP, H), lambda i: (i, 0)),
                  pl.BlockSpec((H, H), lambda i: (0, 0)),
                  pl.BlockSpec((1, H), lambda i: (0, 0))],
        out_specs=pl.BlockSpec((_BLKP,), lambda i: (i,)),
        out_shape=jax.ShapeDtypeStruct((n,), jnp.float32),
    )(x, W, a.reshape(1, H))


def _combine(us, bias_sum, g, bvec, h_prev, n):
    k = len(us)
    has_res = h_prev is not None

    def body(*refs):
        u_refs = refs[:k]
        bs_ref, g_ref, b_ref = refs[k:k + 3]
        res_ref = refs[k + 3] if has_res else None
        out_ref = refs[-1]
        acc = u_refs[0][...]
        for r in u_refs[1:]:
            acc = acc + r[...]
        acc = acc + bs_ref[...]
        mu = jnp.mean(acc, axis=-1, keepdims=True)
        var = jnp.mean((acc - mu) ** 2, axis=-1, keepdims=True)
        y = (acc - mu) * lax.rsqrt(var + _EPS_LN) * g_ref[...] + b_ref[...]
        y = jnp.maximum(y, 0.0)
        if has_res:
            y = y + res_ref[...]
        out_ref[...] = y

    specs = [pl.BlockSpec((_BLKP, H), lambda i: (i, 0)) for _ in range(k)]
    specs += [pl.BlockSpec((1, H), lambda i: (0, 0))] * 3
    args = list(us) + [bias_sum.reshape(1, H), g.reshape(1, H), bvec.reshape(1, H)]
    if has_res:
        specs.append(pl.BlockSpec((_BLKP, H), lambda i: (i, 0)))
        args.append(h_prev)
    return pl.pallas_call(
        body,
        grid=(pl.cdiv(n, _BLKP),),
        in_specs=specs,
        out_specs=pl.BlockSpec((_BLKP, H), lambda i: (i, 0)),
        out_shape=jax.ShapeDtypeStruct((n, H), jnp.float32),
    )(*args)


_MESH = plsc.VectorSubcoreMesh(core_axis_name="c", subcore_axis_name="s")
_NC, _NS, _NW = 2, 16, 32
_CHUNK = 8192


@functools.lru_cache(maxsize=None)
def _pass_a_kernel(ep, ndp, e_real):
    per_tile = ep // _NW
    BLK = 512
    SUB = BLK // 128
    nb = per_tile // BLK
    ZB = 2048
    nzc = ndp // ZB

    @functools.partial(
        pl.kernel,
        out_type=(jax.ShapeDtypeStruct((ep,), jnp.float32),
                  jax.ShapeDtypeStruct((2, ndp), jnp.float32)),
        mesh=_MESH,
        scratch_types=[pltpu.VMEM((3 * SUB, 128), jnp.int32),
                       pltpu.VMEM((3 * SUB, 128), jnp.int32),
                       pltpu.VMEM((2 * SUB, 128), jnp.float32),
                       pltpu.VMEM((2 * SUB, 128), jnp.float32),
                       pltpu.VMEM((2 * SUB, 128), jnp.float32),
                       pltpu.VMEM((ZB,), jnp.float32),
                       pltpu.VMEM_SHARED((ndp,), jnp.float32),
                       pltpu.SemaphoreType.DMA,
                       pltpu.SemaphoreType.DMA,
                       pltpu.SemaphoreType.DMA,
                       pltpu.SemaphoreType.DMA],
    )
    def kern(als_h, ald_h, src_h, dst_h, w_h, s_h,
             src_v, dst_v, asv, adv, wv, zv, s_sh,
             semI, semG, semWL, semWS):
        c = lax.axis_index("c")
        sid = lax.axis_index("s")
        wid = sid * _NC + c

        def zb(i, _):
            zv[pl.ds(i * 16, 16)] = jnp.zeros((16,), jnp.float32)
            return 0
        lax.fori_loop(0, ZB // 16, zb, 0)

        def zs(ch, _):
            @pl.when(sid == lax.rem(ch, _NS))
            def _():
                pltpu.sync_copy(zv, s_sh.at[pl.ds(ch * ZB, ZB)])
            return 0
        lax.fori_loop(0, nzc, zs, 0)
        plsc.subcore_barrier()

        base = wid * per_tile

        def idx_issue(i):
            sl = lax.rem(i, 3) * SUB
            off = base + i * BLK
            for j in range(SUB):
                pltpu.async_copy(src_h.at[pl.ds(off + j * 128, 128)],
                                 src_v.at[sl + j], semI)
                pltpu.async_copy(dst_h.at[pl.ds(off + j * 128, 128)],
                                 dst_v.at[sl + j], semI)

        idx_issue(jnp.int32(0))

        def eb(i, _):
            sl3 = lax.rem(i, 3) * SUB
            sl2 = lax.rem(i, 2) * SUB
            off = base + i * BLK
            for _ in range(2 * SUB):
                pltpu.make_async_copy(src_h.at[pl.ds(0, 128)],
                                      src_v.at[0], semI).wait()

            @pl.when(i >= 2)
            def _():
                for _ in range(SUB):
                    pltpu.make_async_copy(w_h.at[pl.ds(0, 128)],
                                          wv.at[0], semWL).wait()
                    pltpu.make_async_copy(w_h.at[pl.ds(0, 128)],
                                          wv.at[0], semWS).wait()

            @pl.when(i + 1 < nb)
            def _():
                idx_issue(i + 1)

            cps = [pltpu.async_copy(als_h.at[src_v.at[sl3 + j]],
                                    asv.at[sl2 + j], semG)
                   for j in range(SUB)]
            cps += [pltpu.async_copy(ald_h.at[dst_v.at[sl3 + j]],
                                     adv.at[sl2 + j], semG)
                    for j in range(SUB)]
            for cp in cps:
                cp.wait()

            for g in range(BLK // 16):
                r, o16 = g // 8, (g % 8) * 16
                a16 = asv[sl2 + r, pl.ds(o16, 16)] + adv[sl2 + r, pl.ds(o16, 16)]
                al = jnp.where(a16 > 0, a16, a16 * jnp.float32(0.2))
                eidx = lax.iota(jnp.int32, 16) + (off + g * 16)
                w16 = jnp.where(eidx < e_real, jnp.exp(al), jnp.float32(0.0))
                wv[sl2 + r, pl.ds(o16, 16)] = w16
            for j in range(SUB):
                pltpu.async_copy(wv.at[sl2 + j],
                                 w_h.at[pl.ds(off + j * 128, 128)], semWL)
                pltpu.async_copy(wv.at[sl2 + j], s_sh.at[dst_v.at[sl3 + j]],
                                 semWS, add=True)
            return 0
        lax.fori_loop(0, nb, eb, 0)
        for _ in range(2 * SUB):
            pltpu.make_async_copy(w_h.at[pl.ds(0, 128)], wv.at[0], semWL).wait()
            pltpu.make_async_copy(w_h.at[pl.ds(0, 128)], wv.at[0], semWS).wait()
        plsc.subcore_barrier()

        @pl.when(sid == 0)
        def _():
            pltpu.sync_copy(s_sh, s_h.at[c])

    return kern


@functools.lru_cache(maxsize=None)
def _pass_b_kernel(ep, ns, ndp):
    per_tile = ep // _NS
    BLK_B = 512
    nb = per_tile // BLK_B
    cap = 784
    chunk_los = list(range(0, ndp, _CHUNK))

    @functools.partial(
        pl.kernel,
        out_type=jax.ShapeDtypeStruct((ndp, H), jnp.float32),
        mesh=_MESH,
        scratch_types=[pltpu.VMEM((2, BLK_B), jnp.int32),
                       pltpu.VMEM((2, BLK_B), jnp.int32),
                       pltpu.VMEM((2, BLK_B), jnp.float32),
                       pltpu.VMEM((cap,), jnp.int32),
                       pltpu.VMEM((cap,), jnp.int32),
                       pltpu.VMEM((cap,), jnp.float32),
                       pltpu.VMEM((2, 128), jnp.int32),
                       pltpu.VMEM((2, 128, H), jnp.float32),
                       pltpu.VMEM((32, H), jnp.float32),
                       pltpu.VMEM((_CHUNK,), jnp.float32),
                       pltpu.VMEM((_CHUNK,), jnp.float32),
                       pltpu.VMEM_SHARED((_CHUNK, H), jnp.float32),
                       pltpu.SemaphoreType.DMA,
                       pltpu.SemaphoreType.DMA,
                       pltpu.SemaphoreType.DMA],
        compiler_params=pltpu.CompilerParams(needs_layout_passes=False),
    )
    def kern(ps_h, src_h, dst_h, w_h, s0_h, s1_h, u_h,
             dstb, srcb, wb, cdl, csrc, cw, cidx2, rows, zrows,
             s_loc, s_tmp, u_sh, semE, semG, semS):
        c = lax.axis_index("c")
        sid = lax.axis_index("s")

        z16f = jnp.zeros((16,), jnp.float32)
        z16i = jnp.zeros((16,), jnp.int32)

        def zr(i, _):
            zrows[i // 8, pl.ds((i % 8) * 16, 16)] = z16f
            return 0
        lax.fori_loop(0, 32 * 8, zr, 0)

        def g_issue(j):
            sl = lax.rem(j, 2)
            pltpu.async_copy(ps_h.at[csrc.at[pl.ds(j * 128, 128)]],
                             rows.at[sl], semG)

        def g_wait(sl):
            pltpu.make_async_copy(ps_h.at[pl.ds(0, 128)],
                                  rows.at[sl], semG).wait()

        def s_wait(sl):
            pltpu.make_async_copy(ps_h.at[pl.ds(0, 128)],
                                  rows.at[sl], semS).wait()

        def drain(ng):
            @pl.when(ng > 0)
            def _():
                g_issue(jnp.int32(0))

                def floop(j, _):
                    sl = lax.rem(j, 2)
                    be = j * 128
                    g_wait(sl)
                    @pl.when(j >= 1)
                    def _():
                        s_wait(1 - sl)
                    @pl.when(j + 1 < ng)
                    def _():
                        g_issue(j + 1)

                    for g in range(8):
                        cidx2[sl, pl.ds(g * 16, 16)] = (
                            cdl[pl.ds(be + g * 16, 16)])

                    def scale(r, _):
                        ws = cw[pl.ds(be + r, 16)][0]
                        for q in range(8):
                            rows[sl, r, pl.ds(q * 16, 16)] = (
                                rows[sl, r, pl.ds(q * 16, 16)] * ws)
                        return 0
                    lax.fori_loop(0, 128, scale, 0)
                    pltpu.async_copy(rows.at[sl], u_sh.at[cidx2.at[sl]],
                                     semS, add=True)
                    return 0
                lax.fori_loop(0, ng, floop, 0)
                s_wait(lax.rem(ng - 1, 2))

        for ci, lo in enumerate(chunk_los):
            crows = min(_CHUNK, ndp - lo)

            @pl.when(c == ci % 2)
            def _(lo=lo, crows=crows):
                def zloop(z, _):
                    @pl.when(sid == lax.rem(z, _NS))
                    def _():
                        pltpu.sync_copy(zrows, u_sh.at[pl.ds(z * 32, 32)])
                    return 0
                lax.fori_loop(0, crows // 32, zloop, 0)

                pltpu.sync_copy(s0_h.at[pl.ds(lo, crows)],
                                s_loc.at[pl.ds(0, crows)])
                pltpu.sync_copy(s1_h.at[pl.ds(lo, crows)],
                                s_tmp.at[pl.ds(0, crows)])

                def sadd(i, _):
                    s_loc[pl.ds(i * 16, 16)] = (s_loc[pl.ds(i * 16, 16)]
                                                + s_tmp[pl.ds(i * 16, 16)])
                    return 0
                lax.fori_loop(0, crows // 16, sadd, 0)
                plsc.subcore_barrier()

                def e_issue(i):
                    esl = lax.rem(i, 2)
                    off = sid * per_tile + i * BLK_B
                    pltpu.async_copy(dst_h.at[pl.ds(off, BLK_B)],
                                     dstb.at[esl], semE)
                    pltpu.async_copy(src_h.at[pl.ds(off, BLK_B)],
                                     srcb.at[esl], semE)
                    pltpu.async_copy(w_h.at[pl.ds(off, BLK_B)],
                                     wb.at[esl], semE)

                e_issue(jnp.int32(0))

                def blk(i, cnt):
                    esl = lax.rem(i, 2)
                    for _ in range(3):
                        pltpu.make_async_copy(dst_h.at[pl.ds(0, BLK_B)],
                                              dstb.at[0], semE).wait()

                    @pl.when(i + 1 < nb)
                    def _():
                        e_issue(i + 1)

                    def grp(g, cnt):
                        d16 = dstb[esl, pl.ds(g * 16, 16)]
                        s16 = srcb[esl, pl.ds(g * 16, 16)]
                        w16 = wb[esl, pl.ds(g * 16, 16)]
                        m = (d16 >= lo) & (d16 < lo + crows)
                        dl = jnp.clip(d16 - lo, 0, crows - 1)
                        sv = plsc.load_gather(s_loc, [dl])
                        wn = w16 / (sv + jnp.float32(_EPS_SM))
                        plsc.store_compressed(cdl.at[pl.ds(cnt, 16)], dl, mask=m)
                        plsc.store_compressed(csrc.at[pl.ds(cnt, 16)], s16, mask=m)
                        plsc.store_compressed(cw.at[pl.ds(cnt, 16)], wn, mask=m)
                        return cnt + jnp.sum(m.astype(jnp.int32))
                    cnt = lax.fori_loop(0, BLK_B // 16, grp, cnt)

                    ng = cnt // 128
                    drain(ng)
                    sh = ng * 128
                    for t in range(8):
                        cdl[pl.ds(t * 16, 16)] = cdl[pl.ds(sh + t * 16, 16)]
                        csrc[pl.ds(t * 16, 16)] = csrc[pl.ds(sh + t * 16, 16)]
                        cw[pl.ds(t * 16, 16)] = cw[pl.ds(sh + t * 16, 16)]
                    return cnt - sh
                cnt = lax.fori_loop(0, nb, blk, jnp.int32(0))

                for t in range(8):
                    cdl[pl.ds(cnt + t * 16, 16)] = z16i
                    csrc[pl.ds(cnt + t * 16, 16)] = z16i
                    cw[pl.ds(cnt + t * 16, 16)] = z16f
                drain((cnt + 127) // 128)
                plsc.subcore_barrier()

                def dloop(z, _):
                    @pl.when(sid == lax.rem(z, _NS))
                    def _():
                        pltpu.sync_copy(u_sh.at[pl.ds(z * 64, 64)],
                                        u_h.at[pl.ds(lo + z * 64, 64)])
                    return 0
                lax.fori_loop(0, crows // 64, dloop, 0)
                plsc.subcore_barrier()

    return kern


def _prep_edges(ei, n_dst, self_loops):
    src = ei[0].astype(jnp.int32)
    dst = ei[1].astype(jnp.int32)
    if self_loops:
        ar = jnp.arange(n_dst, dtype=jnp.int32)
        src = jnp.concatenate([src, ar])
        dst = jnp.concatenate([dst, ar])
    e = src.shape[0]
    ep = _rup(e, 16384)
    src = jnp.pad(src, (0, ep - e))
    dst = jnp.pad(dst, (0, ep - e))
    return src, dst, e, ep


def _forward_impl(p):
    ndp = {t: _rup(n, 2048) for t, n in NNODES.items()}
    h = {t: _proj_in(p["x_" + t], p["Win_" + t], p["bin_" + t]) for t in NNODES}
    edges = {rel: _prep_edges(p["ei_" + rel], NNODES[d], sl)
             for rel, s, d, sl in RELSPEC}

    for l in range(2):
        us = {t: [] for t in NNODES}
        bsum = {t: jnp.zeros((H,), jnp.float32) for t in NNODES}
        for rel, s, d, sl in RELSPEC:
            W = p[f"W_l{l}_{rel}"]
            a_s = p[f"as_l{l}_{rel}"]
            a_d = p[f"ad_l{l}_{rel}"]
            src, dst, e_real, ep = edges[rel]
            if s == d:
                ps, als, ald = _proj_full(h[s], W, a_s, a_d)
            else:
                ps, als, _ = _proj_full(h[s], W, a_s, a_d)
                ald = _proj_al(h[d], W, a_d)
            w_e, s_pair = _pass_a_kernel(ep, ndp[d], e_real)(
                als, ald, src, dst)
            u = _pass_b_kernel(ep, h[s].shape[0], ndp[d])(
                ps, src, dst, w_e, s_pair[0], s_pair[1])
            us[d].append(u)
            bsum[d] = bsum[d] + p[f"b_l{l}_{rel}"]
        hn = {}
        for t in NNODES:
            hn[t] = _combine(us[t], bsum[t], p[f"lng_l{l}_{t}"],
                             p[f"lnb_l{l}_{t}"], h[t] if l > 0 else None,
                             NNODES[t])
        h = hn
    return (h["cpg"], h["gene"], h["mirna"])


def kernel(x_cpg, x_gene, x_mirna, ei_maps_to, ei_targets, ei_ppi, ei_rev_maps_to, ei_rev_targets, Win_cpg, bin_cpg, Win_gene, bin_gene, Win_mirna, bin_mirna, W_l0_maps_to, as_l0_maps_to, ad_l0_maps_to, b_l0_maps_to, W_l0_targets, as_l0_targets, ad_l0_targets, b_l0_targets, W_l0_ppi, as_l0_ppi, ad_l0_ppi, b_l0_ppi, W_l0_rev_maps_to, as_l0_rev_maps_to, ad_l0_rev_maps_to, b_l0_rev_maps_to, W_l0_rev_targets, as_l0_rev_targets, ad_l0_rev_targets, b_l0_rev_targets, lng_l0_cpg, lnb_l0_cpg, lng_l0_gene, lnb_l0_gene, lng_l0_mirna, lnb_l0_mirna, W_l1_maps_to, as_l1_maps_to, ad_l1_maps_to, b_l1_maps_to, W_l1_targets, as_l1_targets, ad_l1_targets, b_l1_targets, W_l1_ppi, as_l1_ppi, ad_l1_ppi, b_l1_ppi, W_l1_rev_maps_to, as_l1_rev_maps_to, ad_l1_rev_maps_to, b_l1_rev_maps_to, W_l1_rev_targets, as_l1_rev_targets, ad_l1_rev_targets, b_l1_rev_targets, lng_l1_cpg, lnb_l1_cpg, lng_l1_gene, lnb_l1_gene, lng_l1_mirna, lnb_l1_mirna):
    p = dict(locals())
    return _forward_impl(p)


# speedup vs baseline: 1.0403x; 1.0403x over previous
import functools

import jax
import jax.numpy as jnp
from jax import lax
from jax.experimental import pallas as pl
from jax.experimental.pallas import tpu as pltpu
from jax.experimental.pallas import tpu_sc as plsc

H = 128
NNODES = {"cpg": 100000, "gene": 20000, "mirna": 2000}
RELSPEC = [("maps_to", "cpg", "gene", False),
           ("targets", "mirna", "gene", False),
           ("ppi", "gene", "gene", True),
           ("rev_maps_to", "gene", "cpg", False),
           ("rev_targets", "gene", "mirna", False)]

_EPS_SM = 1e-16
_EPS_LN = 1e-5


def _rup(x, m):
    return (x + m - 1) // m * m


_BLKP = 512


def _proj_in(x, W, b):
    n = x.shape[0]

    def body(x_ref, w_ref, b_ref, y_ref):
        y_ref[...] = jnp.dot(x_ref[...], w_ref[...],
                             preferred_element_type=jnp.float32) + b_ref[...]

    return pl.pallas_call(
        body,
        grid=(pl.cdiv(n, _BLKP),),
        in_specs=[pl.BlockSpec((_BLKP, H), lambda i: (i, 0)),
                  pl.BlockSpec((H, H), lambda i: (0, 0)),
                  pl.BlockSpec((1, H), lambda i: (0, 0))],
        out_specs=pl.BlockSpec((_BLKP, H), lambda i: (i, 0)),
        out_shape=jax.ShapeDtypeStruct((n, H), jnp.float32),
    )(x, W, b.reshape(1, H))


def _proj_full(x, W, a_s, a_d):
    n = x.shape[0]

    def body(x_ref, w_ref, as_ref, ad_ref, y_ref, als_ref, ald_ref):
        y = jnp.dot(x_ref[...], w_ref[...], preferred_element_type=jnp.float32)
        y_ref[...] = y
        als_ref[...] = jnp.sum(y * as_ref[...], axis=-1)
        ald_ref[...] = jnp.sum(y * ad_ref[...], axis=-1)

    return pl.pallas_call(
        body,
        grid=(pl.cdiv(n, _BLKP),),
        in_specs=[pl.BlockSpec((_BLKP, H), lambda i: (i, 0)),
                  pl.BlockSpec((H, H), lambda i: (0, 0)),
                  pl.BlockSpec((1, H), lambda i: (0, 0)),
                  pl.BlockSpec((1, H), lambda i: (0, 0))],
        out_specs=[pl.BlockSpec((_BLKP, H), lambda i: (i, 0)),
                   pl.BlockSpec((_BLKP,), lambda i: (i,)),
                   pl.BlockSpec((_BLKP,), lambda i: (i,))],
        out_shape=[jax.ShapeDtypeStruct((n, H), jnp.float32),
                   jax.ShapeDtypeStruct((n,), jnp.float32),
                   jax.ShapeDtypeStruct((n,), jnp.float32)],
    )(x, W, a_s.reshape(1, H), a_d.reshape(1, H))


def _proj_al(x, W, a):
    n = x.shape[0]

    def body(x_ref, w_ref, a_ref, al_ref):
        y = jnp.dot(x_ref[...], w_ref[...], preferred_element_type=jnp.float32)
        al_ref[...] = jnp.sum(y * a_ref[...], axis=-1)

    return pl.pallas_call(
        body,
        grid=(pl.cdiv(n, _BLKP),),
        in_specs=[pl.BlockSpec((_BLKP, H), lambda i: (i, 0)),
                  pl.BlockSpec((H, H), lambda i: (0, 0)),
                  pl.BlockSpec((1, H), lambda i: (0, 0))],
        out_specs=pl.BlockSpec((_BLKP,), lambda i: (i,)),
        out_shape=jax.ShapeDtypeStruct((n,), jnp.float32),
    )(x, W, a.reshape(1, H))


def _combine(u, soffs, bias_sum, g, bvec, h_prev, n):
    k = len(soffs)
    has_res = h_prev is not None

    def body(*refs):
        u_refs = refs[:k]
        bs_ref, g_ref, b_ref = refs[k:k + 3]
        res_ref = refs[k + 3] if has_res else None
        out_ref = refs[-1]
        acc = u_refs[0][...]
        for r in u_refs[1:]:
            acc = acc + r[...]
        acc = acc + bs_ref[...]
        mu = jnp.mean(acc, axis=-1, keepdims=True)
        var = jnp.mean((acc - mu) ** 2, axis=-1, keepdims=True)
        y = (acc - mu) * lax.rsqrt(var + _EPS_LN) * g_ref[...] + b_ref[...]
        y = jnp.maximum(y, 0.0)
        if has_res:
            y = y + res_ref[...]
        out_ref[...] = y

    specs = [pl.BlockSpec((_BLKP, H),
                          lambda i, so=(so // _BLKP): (so + i, 0))
             for so in soffs]
    specs += [pl.BlockSpec((1, H), lambda i: (0, 0))] * 3
    args = [u] * k + [bias_sum.reshape(1, H), g.reshape(1, H),
                      bvec.reshape(1, H)]
    if has_res:
        specs.append(pl.BlockSpec((_BLKP, H), lambda i: (i, 0)))
        args.append(h_prev)
    return pl.pallas_call(
        body,
        grid=(pl.cdiv(n, _BLKP),),
        in_specs=specs,
        out_specs=pl.BlockSpec((_BLKP, H), lambda i: (i, 0)),
        out_shape=jax.ShapeDtypeStruct((n, H), jnp.float32),
    )(*args)


_MESH = plsc.VectorSubcoreMesh(core_axis_name="c", subcore_axis_name="s")
_NC, _NS, _NW = 2, 16, 32
_CHUNK = 6144


@functools.lru_cache(maxsize=None)
def _pass_a_kernel(relcfg, ndtot):
    BLK = 512
    SUB = BLK // 128
    ZB = 2048
    nzc = ndtot // ZB
    nrel = len(relcfg)

    @functools.partial(
        pl.kernel,
        out_type=tuple([jax.ShapeDtypeStruct((ep, ), jnp.float32)
                        for ep, _ in relcfg]
                       + [jax.ShapeDtypeStruct((2, ndtot), jnp.float32)]),
        mesh=_MESH,
        scratch_types=[pltpu.VMEM((3 * SUB, 128), jnp.int32),
                       pltpu.VMEM((3 * SUB, 128), jnp.int32),
                       pltpu.VMEM((2 * SUB, 128), jnp.float32),
                       pltpu.VMEM((2 * SUB, 128), jnp.float32),
                       pltpu.VMEM((2 * SUB, 128), jnp.float32),
                       pltpu.VMEM((ZB,), jnp.float32),
                       pltpu.VMEM_SHARED((ndtot,), jnp.float32),
                       pltpu.SemaphoreType.DMA,
                       pltpu.SemaphoreType.DMA,
                       pltpu.SemaphoreType.DMA,
                       pltpu.SemaphoreType.DMA],
    )
    def kern(*refs):
        als_hs = refs[:nrel]
        ald_h = refs[nrel]
        src_hs = refs[nrel + 1:2 * nrel + 1]
        dst_hs = refs[2 * nrel + 1:3 * nrel + 1]
        w_hs = refs[3 * nrel + 1:4 * nrel + 1]
        s_h = refs[4 * nrel + 1]
        (src_v, dst_v, asv, adv, wv, zv, s_sh,
         semI, semG, semWL, semWS) = refs[4 * nrel + 2:]
        c = lax.axis_index("c")
        sid = lax.axis_index("s")
        wid = sid * _NC + c

        def zb(i, _):
            zv[pl.ds(i * 16, 16)] = jnp.zeros((16,), jnp.float32)
            return 0
        lax.fori_loop(0, ZB // 16, zb, 0)

        def zs(ch, _):
            @pl.when(sid == lax.rem(ch, _NS))
            def _():
                pltpu.sync_copy(zv, s_sh.at[pl.ds(ch * ZB, ZB)])
            return 0
        lax.fori_loop(0, nzc, zs, 0)
        plsc.subcore_barrier()

        for r in range(nrel):
            ep, e_real = relcfg[r]
            als_h, src_h, dst_h, w_h = als_hs[r], src_hs[r], dst_hs[r], w_hs[r]
            per_tile = ep // _NW
            nb = per_tile // BLK
            base = wid * per_tile

            def idx_issue(i):
                sl = lax.rem(i, 3) * SUB
                off = base + i * BLK
                for j in range(SUB):
                    pltpu.async_copy(src_h.at[pl.ds(off + j * 128, 128)],
                                     src_v.at[sl + j], semI)
                    pltpu.async_copy(dst_h.at[pl.ds(off + j * 128, 128)],
                                     dst_v.at[sl + j], semI)

            idx_issue(jnp.int32(0))

            def eb(i, _):
                sl3 = lax.rem(i, 3) * SUB
                sl2 = lax.rem(i, 2) * SUB
                off = base + i * BLK
                for _ in range(2 * SUB):
                    pltpu.make_async_copy(src_h.at[pl.ds(0, 128)],
                                          src_v.at[0], semI).wait()

                @pl.when(i >= 2)
                def _():
                    for _ in range(SUB):
                        pltpu.make_async_copy(w_h.at[pl.ds(0, 128)],
                                              wv.at[0], semWL).wait()
                        pltpu.make_async_copy(w_h.at[pl.ds(0, 128)],
                                              wv.at[0], semWS).wait()

                @pl.when(i + 1 < nb)
                def _():
                    idx_issue(i + 1)

                cps = [pltpu.async_copy(als_h.at[src_v.at[sl3 + j]],
                                        asv.at[sl2 + j], semG)
                       for j in range(SUB)]
                cps += [pltpu.async_copy(ald_h.at[dst_v.at[sl3 + j]],
                                         adv.at[sl2 + j], semG)
                        for j in range(SUB)]
                for cp in cps:
                    cp.wait()

                for g in range(BLK // 16):
                    rr, o16 = g // 8, (g % 8) * 16
                    a16 = (asv[sl2 + rr, pl.ds(o16, 16)]
                           + adv[sl2 + rr, pl.ds(o16, 16)])
                    al = jnp.where(a16 > 0, a16, a16 * jnp.float32(0.2))
                    eidx = lax.iota(jnp.int32, 16) + (off + g * 16)
                    w16 = jnp.where(eidx < e_real, jnp.exp(al),
                                    jnp.float32(0.0))
                    wv[sl2 + rr, pl.ds(o16, 16)] = w16
                for j in range(SUB):
                    pltpu.async_copy(wv.at[sl2 + j],
                                     w_h.at[pl.ds(off + j * 128, 128)], semWL)
                    pltpu.async_copy(wv.at[sl2 + j], s_sh.at[dst_v.at[sl3 + j]],
                                     semWS, add=True)
                return 0
            lax.fori_loop(0, nb, eb, 0)
            for _ in range(2 * SUB):
                pltpu.make_async_copy(w_h.at[pl.ds(0, 128)],
                                      wv.at[0], semWL).wait()
                pltpu.make_async_copy(w_h.at[pl.ds(0, 128)],
                                      wv.at[0], semWS).wait()
        plsc.subcore_barrier()

        @pl.when(sid == 0)
        def _():
            pltpu.sync_copy(s_sh, s_h.at[c])

    return kern


@functools.lru_cache(maxsize=None)
def _pass_b_kernel(relcfg, ndtot):
    BLK_B = 512
    cap = 784
    nrel = len(relcfg)

    @functools.partial(
        pl.kernel,
        out_type=jax.ShapeDtypeStruct((ndtot, H), jnp.float32),
        mesh=_MESH,
        scratch_types=[pltpu.VMEM((2, BLK_B), jnp.int32),
                       pltpu.VMEM((2, BLK_B), jnp.int32),
                       pltpu.VMEM((2, BLK_B), jnp.float32),
                       pltpu.VMEM((cap,), jnp.int32),
                       pltpu.VMEM((cap,), jnp.int32),
                       pltpu.VMEM((cap,), jnp.float32),
                       pltpu.VMEM((2, 128), jnp.int32),
                       pltpu.VMEM((2, 128, H), jnp.float32),
                       pltpu.VMEM((32, H), jnp.float32),
                       pltpu.VMEM((_CHUNK,), jnp.float32),
                       pltpu.VMEM((_CHUNK,), jnp.float32),
                       pltpu.VMEM_SHARED((_CHUNK, H), jnp.float32),
                       pltpu.SemaphoreType.DMA,
                       pltpu.SemaphoreType.DMA,
                       pltpu.SemaphoreType.DMA],
        compiler_params=pltpu.CompilerParams(needs_layout_passes=False),
    )
    def kern(*refs):
        ps_hs = refs[:nrel]
        src_hs = refs[nrel:2 * nrel]
        dst_hs = refs[2 * nrel:3 * nrel]
        w_hs = refs[3 * nrel:4 * nrel]
        s0_h = refs[4 * nrel]
        s1_h = refs[4 * nrel + 1]
        u_h = refs[4 * nrel + 2]
        (dstb, srcb, wb, cdl, csrc, cw, cidx2, rows, zrows,
         s_loc, s_tmp, u_sh, semE, semG, semS) = refs[4 * nrel + 3:]
        c = lax.axis_index("c")
        sid = lax.axis_index("s")

        z16f = jnp.zeros((16,), jnp.float32)
        z16i = jnp.zeros((16,), jnp.int32)

        def zr(i, _):
            zrows[i // 8, pl.ds((i % 8) * 16, 16)] = z16f
            return 0
        lax.fori_loop(0, 32 * 8, zr, 0)

        for r in range(nrel):
            ep, soff, nchr = relcfg[r]
            per_tile = ep // _NS
            nb = per_tile // BLK_B
            ps_h, src_h, dst_h, w_h = ps_hs[r], src_hs[r], dst_hs[r], w_hs[r]

            def g_issue(j, ps_h=ps_h):
                sl = lax.rem(j, 2)
                pltpu.async_copy(ps_h.at[csrc.at[pl.ds(j * 128, 128)]],
                                 rows.at[sl], semG)

            def g_wait(sl, ps_h=ps_h):
                pltpu.make_async_copy(ps_h.at[pl.ds(0, 128)],
                                      rows.at[sl], semG).wait()

            def s_wait(sl, ps_h=ps_h):
                pltpu.make_async_copy(ps_h.at[pl.ds(0, 128)],
                                      rows.at[sl], semS).wait()

            def drain(ng, g_issue=g_issue, g_wait=g_wait, s_wait=s_wait):
                @pl.when(ng > 0)
                def _():
                    g_issue(jnp.int32(0))

                    def floop(j, _):
                        sl = lax.rem(j, 2)
                        be = j * 128
                        g_wait(sl)
                        @pl.when(j >= 1)
                        def _():
                            s_wait(1 - sl)
                        @pl.when(j + 1 < ng)
                        def _():
                            g_issue(j + 1)

                        for g in range(8):
                            cidx2[sl, pl.ds(g * 16, 16)] = (
                                cdl[pl.ds(be + g * 16, 16)])

                        def scale(rr, _):
                            ws = cw[pl.ds(be + rr, 16)][0]
                            for q in range(8):
                                rows[sl, rr, pl.ds(q * 16, 16)] = (
                                    rows[sl, rr, pl.ds(q * 16, 16)] * ws)
                            return 0
                        lax.fori_loop(0, 128, scale, 0)
                        pltpu.async_copy(rows.at[sl], u_sh.at[cidx2.at[sl]],
                                         semS, add=True)
                        return 0
                    lax.fori_loop(0, ng, floop, 0)
                    s_wait(lax.rem(ng - 1, 2))

            def chunk_body(ci, _, src_h=src_h, dst_h=dst_h, w_h=w_h,
                           drain=drain, soff=soff, per_tile=per_tile, nb=nb):
                gi = soff // _CHUNK + ci
                lo = soff + ci * _CHUNK

                @pl.when(c == lax.rem(gi, 2))
                def _():
                    def zloop(z, _):
                        @pl.when(sid == lax.rem(z, _NS))
                        def _():
                            pltpu.sync_copy(zrows, u_sh.at[pl.ds(z * 32, 32)])
                        return 0
                    lax.fori_loop(0, _CHUNK // 32, zloop, 0)

                    pltpu.sync_copy(s0_h.at[pl.ds(lo, _CHUNK)], s_loc)
                    pltpu.sync_copy(s1_h.at[pl.ds(lo, _CHUNK)], s_tmp)

                    def sadd(i, _):
                        s_loc[pl.ds(i * 16, 16)] = (s_loc[pl.ds(i * 16, 16)]
                                                    + s_tmp[pl.ds(i * 16, 16)])
                        return 0
                    lax.fori_loop(0, _CHUNK // 16, sadd, 0)
                    plsc.subcore_barrier()

                    def e_issue(i):
                        esl = lax.rem(i, 2)
                        off = sid * per_tile + i * BLK_B
                        pltpu.async_copy(dst_h.at[pl.ds(off, BLK_B)],
                                         dstb.at[esl], semE)
                        pltpu.async_copy(src_h.at[pl.ds(off, BLK_B)],
                                         srcb.at[esl], semE)
                        pltpu.async_copy(w_h.at[pl.ds(off, BLK_B)],
                                         wb.at[esl], semE)

                    e_issue(jnp.int32(0))

                    def blk(i, cnt):
                        esl = lax.rem(i, 2)
                        for _ in range(3):
                            pltpu.make_async_copy(dst_h.at[pl.ds(0, BLK_B)],
                                                  dstb.at[0], semE).wait()

                        @pl.when(i + 1 < nb)
                        def _():
                            e_issue(i + 1)

                        def grp(g, cnt):
                            d16 = dstb[esl, pl.ds(g * 16, 16)]
                            s16 = srcb[esl, pl.ds(g * 16, 16)]
                            w16 = wb[esl, pl.ds(g * 16, 16)]
                            m = (d16 >= lo) & (d16 < lo + _CHUNK)
                            dl = jnp.clip(d16 - lo, 0, _CHUNK - 1)
                            sv = plsc.load_gather(s_loc, [dl])
                            wn = w16 / (sv + jnp.float32(_EPS_SM))
                            plsc.store_compressed(cdl.at[pl.ds(cnt, 16)],
                                                  dl, mask=m)
                            plsc.store_compressed(csrc.at[pl.ds(cnt, 16)],
                                                  s16, mask=m)
                            plsc.store_compressed(cw.at[pl.ds(cnt, 16)],
                                                  wn, mask=m)
                            return cnt + jnp.sum(m.astype(jnp.int32))
                        cnt = lax.fori_loop(0, BLK_B // 16, grp, cnt)

                        ng = cnt // 128
                        drain(ng)
                        sh = ng * 128
                        for t in range(8):
                            cdl[pl.ds(t * 16, 16)] = cdl[pl.ds(sh + t * 16, 16)]
                            csrc[pl.ds(t * 16, 16)] = (
                                csrc[pl.ds(sh + t * 16, 16)])
                            cw[pl.ds(t * 16, 16)] = cw[pl.ds(sh + t * 16, 16)]
                        return cnt - sh
                    cnt = lax.fori_loop(0, nb, blk, jnp.int32(0))

                    for t in range(8):
                        cdl[pl.ds(cnt + t * 16, 16)] = z16i
                        csrc[pl.ds(cnt + t * 16, 16)] = z16i
                        cw[pl.ds(cnt + t * 16, 16)] = z16f
                    drain((cnt + 127) // 128)
                    plsc.subcore_barrier()

                    def dloop(z, _):
                        @pl.when(sid == lax.rem(z, _NS))
                        def _():
                            pltpu.sync_copy(u_sh.at[pl.ds(z * 64, 64)],
                                            u_h.at[pl.ds(lo + z * 64, 64)])
                        return 0
                    lax.fori_loop(0, _CHUNK // 64, dloop, 0)
                    plsc.subcore_barrier()
                return 0
            lax.fori_loop(0, nchr, chunk_body, 0)

    return kern


def _prep_edges(ei, n_dst, self_loops, soff):
    src = ei[0].astype(jnp.int32)
    dst = ei[1].astype(jnp.int32)
    if self_loops:
        ar = jnp.arange(n_dst, dtype=jnp.int32)
        src = jnp.concatenate([src, ar])
        dst = jnp.concatenate([dst, ar])
    dst = dst + soff
    e = src.shape[0]
    ep = _rup(e, 16384)
    src = jnp.pad(src, (0, ep - e))
    dst = jnp.pad(dst, (0, ep - e), constant_values=soff)
    return src, dst, e, ep


def _forward_impl(p):
    ndps = [_rup(NNODES[d], _CHUNK) for _, _, d, _ in RELSPEC]
    soffs = [sum(ndps[:i]) for i in range(len(ndps))]
    ndtot = sum(ndps)

    h = {t: _proj_in(p["x_" + t], p["Win_" + t], p["bin_" + t]) for t in NNODES}
    edges = [_prep_edges(p["ei_" + rel], NNODES[d], sl, soffs[i])
             for i, (rel, s, d, sl) in enumerate(RELSPEC)]
    acfg = tuple((edges[i][3], edges[i][2]) for i in range(len(RELSPEC)))
    bcfg = tuple((edges[i][3], soffs[i], ndps[i] // _CHUNK)
                 for i in range(len(RELSPEC)))

    for l in range(2):
        als_l, ps_l, ald_parts = [], [], []
        for i, (rel, s, d, sl) in enumerate(RELSPEC):
            W = p[f"W_l{l}_{rel}"]
            a_s = p[f"as_l{l}_{rel}"]
            a_d = p[f"ad_l{l}_{rel}"]
            if s == d:
                ps, als, ald = _proj_full(h[s], W, a_s, a_d)
            else:
                ps, als, _ = _proj_full(h[s], W, a_s, a_d)
                ald = _proj_al(h[d], W, a_d)
            ps_l.append(ps)
            als_l.append(als)
            ald_parts.append(jnp.pad(ald, (0, ndps[i] - ald.shape[0])))
        ald_cat = jnp.concatenate(ald_parts)

        outs = _pass_a_kernel(acfg, ndtot)(
            *als_l, ald_cat,
            *[e[0] for e in edges], *[e[1] for e in edges])
        w_l, s2 = list(outs[:len(RELSPEC)]), outs[len(RELSPEC)]
        u = _pass_b_kernel(bcfg, ndtot)(
            *ps_l, *[e[0] for e in edges], *[e[1] for e in edges], *w_l,
            s2[0], s2[1])

        hn = {}
        for t in NNODES:
            ridx = [i for i, (_, _, d, _) in enumerate(RELSPEC) if d == t]
            bsum = sum(p[f"b_l{l}_{RELSPEC[i][0]}"] for i in ridx)
            hn[t] = _combine(u, [soffs[i] for i in ridx], bsum,
                             p[f"lng_l{l}_{t}"], p[f"lnb_l{l}_{t}"],
                             h[t] if l > 0 else None, NNODES[t])
        h = hn
    return (h["cpg"], h["gene"], h["mirna"])


def kernel(x_cpg, x_gene, x_mirna, ei_maps_to, ei_targets, ei_ppi, ei_rev_maps_to, ei_rev_targets, Win_cpg, bin_cpg, Win_gene, bin_gene, Win_mirna, bin_mirna, W_l0_maps_to, as_l0_maps_to, ad_l0_maps_to, b_l0_maps_to, W_l0_targets, as_l0_targets, ad_l0_targets, b_l0_targets, W_l0_ppi, as_l0_ppi, ad_l0_ppi, b_l0_ppi, W_l0_rev_maps_to, as_l0_rev_maps_to, ad_l0_rev_maps_to, b_l0_rev_maps_to, W_l0_rev_targets, as_l0_rev_targets, ad_l0_rev_targets, b_l0_rev_targets, lng_l0_cpg, lnb_l0_cpg, lng_l0_gene, lnb_l0_gene, lng_l0_mirna, lnb_l0_mirna, W_l1_maps_to, as_l1_maps_to, ad_l1_maps_to, b_l1_maps_to, W_l1_targets, as_l1_targets, ad_l1_targets, b_l1_targets, W_l1_ppi, as_l1_ppi, ad_l1_ppi, b_l1_ppi, W_l1_rev_maps_to, as_l1_rev_maps_to, ad_l1_rev_maps_to, b_l1_rev_maps_to, W_l1_rev_targets, as_l1_rev_targets, ad_l1_rev_targets, b_l1_rev_targets, lng_l1_cpg, lnb_l1_cpg, lng_l1_gene, lnb_l1_gene, lng_l1_mirna, lnb_l1_mirna):
    p = dict(locals())
    return _forward_impl(p)
